# Initial kernel scaffold; baseline (speedup 1.0000x reference)
import functools

import jax
import jax.numpy as jnp
from jax import lax
from jax.experimental import pallas as pl
from jax.experimental.pallas import tpu as pltpu
from jax.experimental.pallas import tpu_sc as plsc

_MAX_SEQ = 4096
_TOTAL = 32768
_BATCH = 16

_NC = 2
_NS = 16
_NW = _NC * _NS
_CHUNK = (_BATCH * _MAX_SEQ) // _NW
_BUF = _CHUNK + 16
_L = 16


def _pad_body(flat_hbm, cu_hbm, out_hbm, mask_hbm, cu_v, buf_v, val_v, msk_v):
    wid = lax.axis_index("s") * _NC + lax.axis_index("c")
    row = wid // 2
    half = wid % 2
    col0 = half * _CHUNK

    pltpu.sync_copy(cu_hbm, cu_v)
    s = cu_v[row]
    e = cu_v[row + 1]
    length = jnp.minimum(e - s, _MAX_SEQ)

    start = s + col0
    start_al = jnp.minimum((start // 8) * 8, _TOTAL - _BUF)
    start_al = pl.multiple_of(start_al, 8)
    off = start - start_al

    pltpu.sync_copy(flat_hbm.at[pl.ds(start_al, _BUF)], buf_v)

    lane = lax.iota(jnp.int32, _L)

    def body(j, carry):
        base = j * _L
        idx = jnp.minimum(off + base + lane, _BUF - 1)
        vals = plsc.load_gather(buf_v, [idx])
        pos = col0 + base + lane
        valid = pos < length
        val_v[pl.ds(base, _L)] = jnp.where(valid, vals, 0.0)
        msk_v[pl.ds(base, _L)] = jnp.where(valid, 1, 0)
        return carry

    lax.fori_loop(0, _CHUNK // _L, body, 0)

    pltpu.sync_copy(val_v, out_hbm.at[row, pl.ds(col0, _CHUNK)])
    pltpu.sync_copy(msk_v, mask_hbm.at[row, pl.ds(col0, _CHUNK)])


_pad_sc = functools.partial(
    pl.kernel,
    out_type=(
        jax.ShapeDtypeStruct((_BATCH, _MAX_SEQ), jnp.float32),
        jax.ShapeDtypeStruct((_BATCH, _MAX_SEQ), jnp.int32),
    ),
    mesh=plsc.VectorSubcoreMesh(core_axis_name="c", subcore_axis_name="s"),
    scratch_types=[
        pltpu.VMEM((32,), jnp.int32),
        pltpu.VMEM((_BUF,), jnp.float32),
        pltpu.VMEM((_CHUNK,), jnp.float32),
        pltpu.VMEM((_CHUNK,), jnp.int32),
    ],
)(_pad_body)


def kernel(flat, cu_seqlens):
    cu32 = jnp.concatenate(
        [cu_seqlens.astype(jnp.int32),
         jnp.zeros((32 - cu_seqlens.shape[0],), jnp.int32)]
    )
    return _pad_sc(flat, cu32)


# speedup vs baseline: 21.9389x; 21.9389x over previous
import functools

import jax
import jax.numpy as jnp
from jax import lax
from jax.experimental import pallas as pl
from jax.experimental.pallas import tpu as pltpu
from jax.experimental.pallas import tpu_sc as plsc

_MAX_SEQ = 4096
_TOTAL = 32768
_BATCH = 16

_NC = 2
_NS = 16
_NW = _NC * _NS
_CHUNK = (_BATCH * _MAX_SEQ) // _NW
_BUF = _CHUNK + 16
_L = 16


def _pad_body(flat_hbm, cu_hbm, out_hbm, mask_hbm, cu_v, buf_v, val_v, msk_v):
    wid = lax.axis_index("s") * _NC + lax.axis_index("c")
    row = wid // 2
    half = wid % 2
    col0 = half * _CHUNK

    pltpu.sync_copy(cu_hbm, cu_v)
    cu_pair = cu_v[pl.ds(row, _L)]
    s = cu_pair[0]
    e = cu_pair[1]
    length = jnp.minimum(e - s, _MAX_SEQ)

    start = s + col0
    start_al = jnp.minimum((start // 8) * 8, _TOTAL - _BUF)
    start_al = pl.multiple_of(start_al, 8)
    off = start - start_al

    pltpu.sync_copy(flat_hbm.at[pl.ds(start_al, _BUF)], buf_v)

    lane = lax.iota(jnp.int32, _L)

    def body(j, carry):
        base = j * _L
        idx = jnp.minimum(off + base + lane, _BUF - 1)
        vals = plsc.load_gather(buf_v, [idx])
        pos = col0 + base + lane
        valid = pos < length
        val_v[pl.ds(base, _L)] = jnp.where(valid, vals, 0.0)
        msk_v[pl.ds(base, _L)] = jnp.where(valid, 1, 0)
        return carry

    lax.fori_loop(0, _CHUNK // _L, body, 0)

    pltpu.sync_copy(val_v, out_hbm.at[row, pl.ds(col0, _CHUNK)])
    pltpu.sync_copy(msk_v, mask_hbm.at[row, pl.ds(col0, _CHUNK)])


_pad_sc = functools.partial(
    pl.kernel,
    out_type=(
        jax.ShapeDtypeStruct((_BATCH, _MAX_SEQ), jnp.float32),
        jax.ShapeDtypeStruct((_BATCH, _MAX_SEQ), jnp.int32),
    ),
    mesh=plsc.VectorSubcoreMesh(core_axis_name="c", subcore_axis_name="s"),
    compiler_params=pltpu.CompilerParams(needs_layout_passes=False),
    scratch_types=[
        pltpu.VMEM((32,), jnp.int32),
        pltpu.VMEM((_BUF,), jnp.float32),
        pltpu.VMEM((_CHUNK,), jnp.float32),
        pltpu.VMEM((_CHUNK,), jnp.int32),
    ],
)(_pad_body)


def kernel(flat, cu_seqlens):
    cu32 = jnp.concatenate(
        [cu_seqlens.astype(jnp.int32),
         jnp.zeros((32 - cu_seqlens.shape[0],), jnp.int32)]
    )
    return _pad_sc(flat, cu32)
